# baseline (speedup 1.0000x reference)
import math

import jax
import jax.numpy as jnp
import numpy as np
from jax import lax
from jax.experimental import pallas as pl
from jax.experimental.pallas import tpu as pltpu
from jax.experimental.pallas import tpu_sc as plsc

NUM_VOCAB = 1000000
EMBED_DIM = 64
MAXLEN = 512
BATCH = 1024
SEQLEN = 200

NC = 2
NS = 16
NW = NC * NS

TOTAL = BATCH * SEQLEN
CHUNK = 128
PER_WORKER = TOTAL // NW
CHUNKS_PER_WORKER = PER_WORKER // CHUNK
PAIR_SEQ = SEQLEN // 2
VEC = 16


def _make_pe(maxlen, d_model):
    position = np.arange(maxlen, dtype=np.float32)[:, None]
    div_term = np.exp(
        np.arange(0, d_model, 2).astype(np.float32) * (-math.log(10000.0) / d_model)
    )
    pe = np.zeros((maxlen, d_model), dtype=np.float32)
    pe[:, 0::2] = np.sin(position * div_term)
    pe[:, 1::2] = np.cos(position * div_term)
    return pe


_PE_PAIR = _make_pe(MAXLEN, EMBED_DIM)[:SEQLEN].reshape(PAIR_SEQ, 2 * EMBED_DIM)


def _sc_embed(idx_flat, table2, pe):
    mesh = plsc.VectorSubcoreMesh(core_axis_name="c", subcore_axis_name="s")

    @pl.kernel(
        out_type=jax.ShapeDtypeStruct((TOTAL // 2, 2 * EMBED_DIM), jnp.float32),
        mesh=mesh,
        compiler_params=pltpu.CompilerParams(needs_layout_passes=False),
        scratch_types=[
            pltpu.VMEM((PER_WORKER,), jnp.int32),
            pltpu.VMEM((PER_WORKER,), jnp.int32),
            pltpu.VMEM((PAIR_SEQ, 2 * EMBED_DIM), jnp.float32),
            pltpu.VMEM((CHUNK, 2 * EMBED_DIM), jnp.float32),
            pltpu.VMEM((CHUNK // 2, 2 * EMBED_DIM), jnp.float32),
            pltpu.SemaphoreType.DMA,
        ],
    )
    def k(idx_hbm, table_hbm, pe_hbm, out_hbm, idx_v, q_v, pe_v, g, o, sem):
        wid = lax.axis_index("s") * NC + lax.axis_index("c")
        pltpu.sync_copy(idx_hbm.at[pl.ds(wid * PER_WORKER, PER_WORKER)], idx_v)
        pltpu.sync_copy(pe_hbm, pe_v)

        def pair_idx_body(m, carry):
            sl = pl.ds(m * VEC, VEC)
            q_v[sl] = lax.shift_right_logical(idx_v[sl], 1)
            return carry

        lax.fori_loop(0, PER_WORKER // VEC, pair_idx_body, 0)

        iota = lax.iota(jnp.int32, VEC)

        def chunk_body(j, p0):
            pltpu.async_copy(
                table_hbm.at[q_v.at[pl.ds(j * CHUNK, CHUNK)]], g, sem
            ).wait()
            cbase = j * CHUNK

            def pair_body(kk, p):
                for half in range(2):
                    i = 2 * kk + half
                    t_b = plsc.load_gather(
                        idx_v, [jnp.full((VEC,), cbase + i, jnp.int32)]
                    )
                    t_i = plsc.bitcast(t_b, jnp.int32)
                    h64 = lax.shift_left((t_i & 1), 6)
                    row = jnp.full((VEC,), i, jnp.int32)
                    for d in range(EMBED_DIM // VEC):
                        col = h64 + (d * VEC + iota)
                        v = plsc.load_gather(g, [row, col])
                        sl = pl.ds(half * EMBED_DIM + d * VEC, VEC)
                        o[kk, sl] = v * 8.0 + pe_v[p, sl]
                nxt = p + 1
                return lax.select(nxt == PAIR_SEQ, 0, nxt)

            p_end = lax.fori_loop(0, CHUNK // 2, pair_body, p0)
            base = (wid * CHUNKS_PER_WORKER + j) * (CHUNK // 2)
            pltpu.sync_copy(o, out_hbm.at[pl.ds(base, CHUNK // 2)])
            return p_end

        lax.fori_loop(0, CHUNKS_PER_WORKER, chunk_body, 0)

    return k(idx_flat, table2, pe)


def kernel(tokens, table):
    idx_flat = tokens.reshape(TOTAL).astype(jnp.int32)
    table2 = table.reshape(NUM_VOCAB // 2, 2 * EMBED_DIM)
    out = _sc_embed(idx_flat, table2, jnp.asarray(_PE_PAIR))
    return out.reshape(BATCH, SEQLEN, EMBED_DIM)
